# baseline (speedup 1.0000x reference)
import math

import jax
import jax.numpy as jnp
from jax import lax
from jax.experimental import pallas as pl
from jax.experimental.pallas import tpu as pltpu
from jax.experimental.pallas import tpu_sc as plsc

D_EMBED = 64
SCALE = math.sqrt(D_EMBED)

NUM_CORES = 2
NUM_SUBCORES = 16
NUM_WORKERS = NUM_CORES * NUM_SUBCORES
SEG = 128
NBUF = 4


def _make_kernel(rows_per_tile, seq_len):
    seg_sizes = (SEG, seq_len - SEG)
    assert 0 < seg_sizes[1] <= SEG and seg_sizes[1] % 8 == 0
    n_units = rows_per_tile * 2
    assert n_units % NBUF == 0 and NBUF % 2 == 0
    n_groups = n_units // NBUF

    mesh = plsc.VectorSubcoreMesh(
        core_axis_name="c", subcore_axis_name="s",
        num_cores=NUM_CORES, num_subcores=NUM_SUBCORES)

    def unit_args(u_static, g):
        seg = u_static % 2
        row = g * (NBUF // 2) + u_static // 2
        return row, seg * SEG, seg_sizes[seg]

    def body(x_hbm, table_hbm, out_hbm, idx_v, gbuf, obuf, *sems):
        gsems = sems[:NBUF]
        osems = sems[NBUF:]
        wid = lax.axis_index("s") * NUM_CORES + lax.axis_index("c")
        row0 = wid * rows_per_tile
        pltpu.sync_copy(x_hbm.at[pl.ds(row0, rows_per_tile)], idx_v)

        def start_gather(b, row, coff, ln):
            pltpu.async_copy(
                table_hbm.at[idx_v.at[row, pl.ds(coff, ln)]],
                gbuf.at[b, pl.ds(0, ln)], gsems[b])

        for b in range(NBUF):
            row, coff, ln = unit_args(b, 0)
            start_gather(b, row, coff, ln)

        def group_body(g, _):
            for b in range(NBUF):
                row, coff, ln = unit_args(b, g)
                pltpu.make_async_copy(
                    table_hbm.at[idx_v.at[row, pl.ds(coff, ln)]],
                    gbuf.at[b, pl.ds(0, ln)], gsems[b]).wait()

                @pl.when(g > 0)
                def _():
                    pltpu.make_async_copy(
                        obuf.at[b, pl.ds(0, ln)],
                        out_hbm.at[row, pl.ds(coff, ln)], osems[b]).wait()

                @plsc.parallel_loop(0, ln, unroll=4)
                def _(r):
                    for c in range(D_EMBED // 16):
                        sl = pl.ds(c * 16, 16)
                        obuf[b, r, sl] = gbuf[b, r, sl] * SCALE

                @pl.when(g + 1 < n_groups)
                def _():
                    nrow, ncoff, nln = unit_args(b, g + 1)
                    start_gather(b, nrow, ncoff, nln)

                pltpu.async_copy(
                    obuf.at[b, pl.ds(0, ln)],
                    out_hbm.at[row, pl.ds(coff, ln)], osems[b])
            return 0

        lax.fori_loop(0, n_groups, group_body, 0)

        for b in range(NBUF):
            row, coff, ln = unit_args(b, n_groups - 1)
            pltpu.make_async_copy(
                obuf.at[b, pl.ds(0, ln)],
                out_hbm.at[row, pl.ds(coff, ln)], osems[b]).wait()

    return pl.kernel(
        body,
        out_type=jax.ShapeDtypeStruct(
            (NUM_WORKERS * rows_per_tile, seq_len, D_EMBED), jnp.float32),
        mesh=mesh,
        scratch_types=[
            pltpu.VMEM((rows_per_tile, seq_len), jnp.int32),
            pltpu.VMEM((NBUF, SEG, D_EMBED), jnp.float32),
            pltpu.VMEM((NBUF, SEG, D_EMBED), jnp.float32),
        ] + [pltpu.SemaphoreType.DMA] * (2 * NBUF),
        compiler_params=pltpu.CompilerParams(use_tc_tiling_on_sc=False),
    )


def kernel(x, table):
    b, s = x.shape
    assert b % NUM_WORKERS == 0
    rows_per_tile = b // NUM_WORKERS
    return _make_kernel(rows_per_tile, s)(x.astype(jnp.int32), table)


# speedup vs baseline: 1.0624x; 1.0624x over previous
import math

import jax
import jax.numpy as jnp
from jax import lax
from jax.experimental import pallas as pl
from jax.experimental.pallas import tpu as pltpu
from jax.experimental.pallas import tpu_sc as plsc

D_EMBED = 64
SCALE = math.sqrt(D_EMBED)

NUM_CORES = 2
NUM_SUBCORES = 16
NUM_WORKERS = NUM_CORES * NUM_SUBCORES
SEG = 128
NBUF = 4


def _make_kernel(rows_per_tile, seq_len):
    seg_sizes = (SEG, seq_len - SEG)
    assert 0 < seg_sizes[1] <= SEG and seg_sizes[1] % 8 == 0
    assert seq_len % 8 == 0
    n_units = rows_per_tile * 2
    assert n_units % NBUF == 0 and NBUF % 2 == 0
    n_groups = n_units // NBUF

    mesh = plsc.VectorSubcoreMesh(
        core_axis_name="c", subcore_axis_name="s",
        num_cores=NUM_CORES, num_subcores=NUM_SUBCORES)

    def unit_args(u_static, g):
        seg = u_static % 2
        row = g * (NBUF // 2) + u_static // 2
        return row, seg * SEG, seg_sizes[seg]

    def body(x_hbm, table_hbm, out_hbm, idx_v, gbuf, obuf, *sems):
        isem = sems[0]
        gsems = sems[1:1 + NBUF]
        osems = sems[1 + NBUF:]
        wid = lax.axis_index("s") * NUM_CORES + lax.axis_index("c")
        row0 = wid * rows_per_tile

        for r in range(rows_per_tile):
            pltpu.async_copy(
                x_hbm.at[row0 + r], idx_v.at[pl.ds(r * seq_len, seq_len)],
                isem)
        for r in range(rows_per_tile):
            pltpu.make_async_copy(
                x_hbm.at[row0 + r], idx_v.at[pl.ds(r * seq_len, seq_len)],
                isem).wait()

        def idx_slice(row, coff, ln):
            off = pl.multiple_of(row * seq_len + coff, 8)
            return idx_v.at[pl.ds(off, ln)]

        def start_gather(b, row, coff, ln):
            pltpu.async_copy(
                table_hbm.at[idx_slice(row, coff, ln)],
                gbuf.at[b, pl.ds(0, ln)], gsems[b])

        for b in range(NBUF):
            row, coff, ln = unit_args(b, 0)
            start_gather(b, row, coff, ln)

        def group_body(g, _):
            for b in range(NBUF):
                row, coff, ln = unit_args(b, g)
                pltpu.make_async_copy(
                    table_hbm.at[idx_slice(row, coff, ln)],
                    gbuf.at[b, pl.ds(0, ln)], gsems[b]).wait()

                @pl.when(g > 0)
                def _():
                    pltpu.make_async_copy(
                        obuf.at[b, pl.ds(0, ln)],
                        out_hbm.at[row0 + row, pl.ds(coff, ln)], osems[b]).wait()

                @plsc.parallel_loop(0, ln, unroll=4)
                def _(r):
                    for c in range(D_EMBED // 16):
                        sl = pl.ds(c * 16, 16)
                        obuf[b, r, sl] = gbuf[b, r, sl] * SCALE

                @pl.when(g + 1 < n_groups)
                def _():
                    nrow, ncoff, nln = unit_args(b, g + 1)
                    start_gather(b, nrow, ncoff, nln)

                pltpu.async_copy(
                    obuf.at[b, pl.ds(0, ln)],
                    out_hbm.at[row0 + row, pl.ds(coff, ln)], osems[b])
            return 0

        lax.fori_loop(0, n_groups, group_body, 0)

        for b in range(NBUF):
            row, coff, ln = unit_args(b, n_groups - 1)
            pltpu.make_async_copy(
                obuf.at[b, pl.ds(0, ln)],
                out_hbm.at[row0 + row, pl.ds(coff, ln)], osems[b]).wait()

    return pl.kernel(
        body,
        out_type=jax.ShapeDtypeStruct(
            (NUM_WORKERS * rows_per_tile, seq_len, D_EMBED), jnp.float32),
        mesh=mesh,
        scratch_types=[
            pltpu.VMEM((rows_per_tile * seq_len,), jnp.int32),
            pltpu.VMEM((NBUF, SEG, D_EMBED), jnp.float32),
            pltpu.VMEM((NBUF, SEG, D_EMBED), jnp.float32),
        ] + [pltpu.SemaphoreType.DMA] * (1 + 2 * NBUF),
        compiler_params=pltpu.CompilerParams(use_tc_tiling_on_sc=False),
    )


def kernel(x, table):
    b, s = x.shape
    assert b % NUM_WORKERS == 0
    rows_per_tile = b // NUM_WORKERS
    return _make_kernel(rows_per_tile, s)(x.astype(jnp.int32), table)
